# baseline (speedup 1.0000x reference)
import functools

import jax
import jax.numpy as jnp
from jax import lax
from jax.experimental import pallas as pl
from jax.experimental.pallas import tpu as pltpu
from jax.experimental.pallas import tpu_sc as plsc

A = 4096
S = 50
V = 100000
D = 64
NW = 32
L = 16

_mesh = plsc.VectorSubcoreMesh(core_axis_name="c", subcore_axis_name="s")


@functools.partial(
    pl.kernel,
    mesh=_mesh,
    out_type=jax.ShapeDtypeStruct((S, D, A), jnp.float32),
    compiler_params=pltpu.CompilerParams(needs_layout_passes=False),
    scratch_types=[
        pltpu.VMEM((V,), jnp.float32),
        pltpu.VMEM((A,), jnp.int32),
        pltpu.VMEM((A,), jnp.float32),
        pltpu.SemaphoreType.DMA,
    ],
)
def _gather_kernel(idx_hbm, tablet_hbm, out_hbm, col_v, idx_v, val_v, sem):
    wid = lax.axis_index("s") * 2 + lax.axis_index("c")

    for dpass in range(2):
        d = wid + NW * dpass
        pltpu.sync_copy(tablet_hbm.at[d], col_v)

        @pl.loop(0, S)
        def _slot(b):
            pltpu.sync_copy(idx_hbm.at[pl.ds(b * A, A)], idx_v)

            @pl.loop(0, A // L)
            def _grp(j):
                idx16 = idx_v[pl.ds(j * L, L)]
                val_v[pl.ds(j * L, L)] = plsc.load_gather(col_v, [idx16])

            pltpu.sync_copy(val_v, out_hbm.at[b, d])


def kernel(items, table):
    idx = items.T.reshape(-1)
    out = _gather_kernel(idx, table.T)
    return out.transpose(2, 0, 1)


# speedup vs baseline: 2.5504x; 2.5504x over previous
import functools

import jax
import jax.numpy as jnp
from jax import lax
from jax.experimental import pallas as pl
from jax.experimental.pallas import tpu as pltpu
from jax.experimental.pallas import tpu_sc as plsc

A = 4096
S = 50
V = 100000
D = 64
NW = 32
L = 16

_mesh = plsc.VectorSubcoreMesh(core_axis_name="c", subcore_axis_name="s")


@functools.partial(
    pl.kernel,
    mesh=_mesh,
    out_type=jax.ShapeDtypeStruct((S, D, A), jnp.float32),
    compiler_params=pltpu.CompilerParams(needs_layout_passes=False),
    scratch_types=[
        pltpu.VMEM((V,), jnp.float32),
        pltpu.VMEM((A,), jnp.int32),
        pltpu.VMEM((A,), jnp.int32),
        pltpu.VMEM((A,), jnp.float32),
        pltpu.VMEM((A,), jnp.float32),
        pltpu.SemaphoreType.DMA,
        pltpu.SemaphoreType.DMA,
        pltpu.SemaphoreType.DMA,
        pltpu.SemaphoreType.DMA,
    ],
)
def _gather_kernel(idx_hbm, tablet_hbm, out_hbm, col_v, idx_v0, idx_v1,
                   val_v0, val_v1, isem0, isem1, ssem0, ssem1):
    wid = lax.axis_index("s") * 2 + lax.axis_index("c")
    slots = ((idx_v0, val_v0, isem0, ssem0), (idx_v1, val_v1, isem1, ssem1))

    for dpass in range(2):
        d = wid + NW * dpass
        pltpu.sync_copy(tablet_hbm.at[d], col_v)

        for slot in range(2):
            iv, _, isem, _ = slots[slot]
            pltpu.async_copy(idx_hbm.at[pl.ds(slot * A, A)], iv, isem)

        @pl.loop(0, S, step=2)
        def _pair(b0):
            for slot in range(2):
                iv, vv, isem, ssem = slots[slot]
                b = b0 + slot
                pltpu.make_async_copy(idx_hbm.at[pl.ds(b * A, A)], iv,
                                      isem).wait()

                @pl.when(b0 >= 2)
                def _():
                    pltpu.make_async_copy(vv, out_hbm.at[b - 2, d],
                                          ssem).wait()

                @plsc.parallel_loop(0, A // L, unroll=8)
                def _grp(j):
                    idx16 = iv[pl.ds(j * L, L)]
                    vv[pl.ds(j * L, L)] = plsc.load_gather(col_v, [idx16])

                pltpu.async_copy(vv, out_hbm.at[b, d], ssem)

                @pl.when(b + 2 < S)
                def _():
                    pltpu.async_copy(idx_hbm.at[pl.ds((b + 2) * A, A)], iv,
                                     isem)

        for slot in range(2):
            _, vv, _, ssem = slots[slot]
            pltpu.make_async_copy(vv, out_hbm.at[S - 2 + slot, d],
                                  ssem).wait()


def kernel(items, table):
    idx = items.T.reshape(-1)
    out = _gather_kernel(idx, table.T)
    return out.transpose(2, 0, 1)


# speedup vs baseline: 2.5584x; 1.0031x over previous
import functools

import jax
import jax.numpy as jnp
from jax import lax
from jax.experimental import pallas as pl
from jax.experimental.pallas import tpu as pltpu
from jax.experimental.pallas import tpu_sc as plsc

A = 4096
S = 50
V = 100000
D = 64
NW = 32
L = 16

_mesh = plsc.VectorSubcoreMesh(core_axis_name="c", subcore_axis_name="s")


@functools.partial(
    pl.kernel,
    mesh=_mesh,
    out_type=jax.ShapeDtypeStruct((S, D, A), jnp.float32),
    compiler_params=pltpu.CompilerParams(needs_layout_passes=False),
    scratch_types=[
        pltpu.VMEM((V,), jnp.float32),
        pltpu.VMEM((A,), jnp.int32),
        pltpu.VMEM((A,), jnp.int32),
        pltpu.VMEM((A,), jnp.float32),
        pltpu.VMEM((A,), jnp.float32),
        pltpu.SemaphoreType.DMA,
        pltpu.SemaphoreType.DMA,
        pltpu.SemaphoreType.DMA,
        pltpu.SemaphoreType.DMA,
    ],
)
def _gather_kernel(idx_hbm, tablet_hbm, out_hbm, col_v, idx_v0, idx_v1,
                   val_v0, val_v1, isem0, isem1, ssem0, ssem1):
    wid = lax.axis_index("s") * 2 + lax.axis_index("c")
    slots = ((idx_v0, val_v0, isem0, ssem0), (idx_v1, val_v1, isem1, ssem1))

    for dpass in range(2):
        d = wid + NW * dpass
        pltpu.sync_copy(tablet_hbm.at[d], col_v)

        for slot in range(2):
            iv, _, isem, _ = slots[slot]
            pltpu.async_copy(idx_hbm.at[pl.ds(slot * A, A)], iv, isem)

        @pl.loop(0, S, step=2)
        def _pair(b0):
            for slot in range(2):
                iv, vv, isem, ssem = slots[slot]
                b = b0 + slot
                pltpu.make_async_copy(idx_hbm.at[pl.ds(b * A, A)], iv,
                                      isem).wait()

                @pl.when(b0 >= 2)
                def _():
                    pltpu.make_async_copy(vv, out_hbm.at[b - 2, d],
                                          ssem).wait()

                @plsc.parallel_loop(0, A // L, unroll=16)
                def _grp(j):
                    idx16 = iv[pl.ds(j * L, L)]
                    vv[pl.ds(j * L, L)] = plsc.load_gather(col_v, [idx16])

                pltpu.async_copy(vv, out_hbm.at[b, d], ssem)

                @pl.when(b + 2 < S)
                def _():
                    pltpu.async_copy(idx_hbm.at[pl.ds((b + 2) * A, A)], iv,
                                     isem)

        for slot in range(2):
            _, vv, _, ssem = slots[slot]
            pltpu.make_async_copy(vv, out_hbm.at[S - 2 + slot, d],
                                  ssem).wait()


def kernel(items, table):
    idx = items.T.reshape(-1)
    out = _gather_kernel(idx, table.T)
    return out.transpose(2, 0, 1)


# speedup vs baseline: 4.3295x; 1.6923x over previous
import functools

import jax
import jax.numpy as jnp
from jax import lax
from jax.experimental import pallas as pl
from jax.experimental.pallas import tpu as pltpu
from jax.experimental.pallas import tpu_sc as plsc

A = 4096
S = 50
V = 100000
D = 64
NW = 32
L = 16

_mesh = plsc.VectorSubcoreMesh(core_axis_name="c", subcore_axis_name="s")


@functools.partial(
    pl.kernel,
    mesh=_mesh,
    out_type=jax.ShapeDtypeStruct((S, D, A), jnp.float32),
    compiler_params=pltpu.CompilerParams(needs_layout_passes=False),
    scratch_types=[
        pltpu.VMEM((V,), jnp.float32),
        pltpu.VMEM((A,), jnp.int32),
        pltpu.VMEM((A,), jnp.int32),
        pltpu.VMEM((A,), jnp.float32),
        pltpu.VMEM((A,), jnp.float32),
        pltpu.VMEM_SHARED((S * A,), jnp.int32),
        pltpu.SemaphoreType.DMA,
        pltpu.SemaphoreType.DMA,
        pltpu.SemaphoreType.DMA,
        pltpu.SemaphoreType.DMA,
    ],
)
def _gather_kernel(idx_hbm, tablet_hbm, out_hbm, col_v, idx_v0, idx_v1,
                   val_v0, val_v1, idx_sh, isem0, isem1, ssem0, ssem1):
    wid = lax.axis_index("s") * 2 + lax.axis_index("c")
    slots = ((idx_v0, val_v0, isem0, ssem0), (idx_v1, val_v1, isem1, ssem1))

    @pl.when(lax.axis_index("s") == 0)
    def _():
        pltpu.sync_copy(idx_hbm, idx_sh)

    plsc.subcore_barrier()

    for dpass in range(2):
        d = wid + NW * dpass
        pltpu.sync_copy(tablet_hbm.at[d], col_v)

        for slot in range(2):
            iv, _, isem, _ = slots[slot]
            pltpu.async_copy(idx_sh.at[pl.ds(slot * A, A)], iv, isem)

        @pl.loop(0, S, step=2)
        def _pair(b0):
            for slot in range(2):
                iv, vv, isem, ssem = slots[slot]
                b = b0 + slot
                pltpu.make_async_copy(idx_sh.at[pl.ds(b * A, A)], iv,
                                      isem).wait()

                @pl.when(b0 >= 2)
                def _():
                    pltpu.make_async_copy(vv, out_hbm.at[b - 2, d],
                                          ssem).wait()

                @plsc.parallel_loop(0, A // L, unroll=16)
                def _grp(j):
                    idx16 = iv[pl.ds(j * L, L)]
                    vv[pl.ds(j * L, L)] = plsc.load_gather(col_v, [idx16])

                pltpu.async_copy(vv, out_hbm.at[b, d], ssem)

                @pl.when(b + 2 < S)
                def _():
                    pltpu.async_copy(idx_sh.at[pl.ds((b + 2) * A, A)], iv,
                                     isem)

        for slot in range(2):
            _, vv, _, ssem = slots[slot]
            pltpu.make_async_copy(vv, out_hbm.at[S - 2 + slot, d],
                                  ssem).wait()


def kernel(items, table):
    idx = items.T.reshape(-1)
    out = _gather_kernel(idx, table.T)
    return out.transpose(2, 0, 1)


# speedup vs baseline: 4.4170x; 1.0202x over previous
import functools

import jax
import jax.numpy as jnp
from jax import lax
from jax.experimental import pallas as pl
from jax.experimental.pallas import tpu as pltpu
from jax.experimental.pallas import tpu_sc as plsc

A = 4096
S = 50
V = 100000
D = 64
NW = 32
L = 16

_mesh = plsc.VectorSubcoreMesh(core_axis_name="c", subcore_axis_name="s")


@functools.partial(
    pl.kernel,
    mesh=_mesh,
    out_type=jax.ShapeDtypeStruct((S, D, A), jnp.float32),
    compiler_params=pltpu.CompilerParams(needs_layout_passes=False),
    scratch_types=[
        pltpu.VMEM((V,), jnp.float32),
        pltpu.VMEM((A,), jnp.int32),
        pltpu.VMEM((A,), jnp.int32),
        pltpu.VMEM((A,), jnp.float32),
        pltpu.VMEM((A,), jnp.float32),
        pltpu.VMEM_SHARED((S * A,), jnp.int32),
        pltpu.SemaphoreType.DMA,
        pltpu.SemaphoreType.DMA,
        pltpu.SemaphoreType.DMA,
        pltpu.SemaphoreType.DMA,
        pltpu.SemaphoreType.DMA,
    ],
)
def _gather_kernel(idx_hbm, tablet_hbm, out_hbm, col_v, idx_v0, idx_v1,
                   val_v0, val_v1, idx_sh, isem0, isem1, ssem0, ssem1, csem):
    sid = lax.axis_index("s")
    wid = sid * 2 + lax.axis_index("c")
    slots = ((idx_v0, val_v0, isem0, ssem0), (idx_v1, val_v1, isem1, ssem1))

    col0 = pltpu.async_copy(tablet_hbm.at[wid], col_v, csem)

    stage = (S * A) // 16
    pltpu.sync_copy(idx_hbm.at[pl.ds(sid * stage, stage)],
                    idx_sh.at[pl.ds(sid * stage, stage)])
    plsc.subcore_barrier()

    for dpass in range(2):
        d = wid + NW * dpass
        if dpass == 0:
            col0.wait()
        else:
            pltpu.sync_copy(tablet_hbm.at[d], col_v)

        for slot in range(2):
            iv, _, isem, _ = slots[slot]
            pltpu.async_copy(idx_sh.at[pl.ds(slot * A, A)], iv, isem)

        @pl.loop(0, S, step=2)
        def _pair(b0):
            for slot in range(2):
                iv, vv, isem, ssem = slots[slot]
                b = b0 + slot
                pltpu.make_async_copy(idx_sh.at[pl.ds(b * A, A)], iv,
                                      isem).wait()

                @pl.when(b0 >= 2)
                def _():
                    pltpu.make_async_copy(vv, out_hbm.at[b - 2, d],
                                          ssem).wait()

                @plsc.parallel_loop(0, A // L, unroll=16)
                def _grp(j):
                    idx16 = iv[pl.ds(j * L, L)]
                    vv[pl.ds(j * L, L)] = plsc.load_gather(col_v, [idx16])

                pltpu.async_copy(vv, out_hbm.at[b, d], ssem)

                @pl.when(b + 2 < S)
                def _():
                    pltpu.async_copy(idx_sh.at[pl.ds((b + 2) * A, A)], iv,
                                     isem)

        for slot in range(2):
            _, vv, _, ssem = slots[slot]
            pltpu.make_async_copy(vv, out_hbm.at[S - 2 + slot, d],
                                  ssem).wait()


def kernel(items, table):
    idx = items.T.reshape(-1)
    out = _gather_kernel(idx, table.T)
    return out.transpose(2, 0, 1)
